# baseline (speedup 1.0000x reference)
import functools

import jax
import jax.numpy as jnp
from jax import lax
from jax.experimental import pallas as pl
from jax.experimental.pallas import tpu as pltpu
from jax.experimental.pallas import tpu_sc as plsc

LN_EPS = 1e-7
NC = 2
NS = 16
NW = NC * NS
CH = 32
NBUF = 4


def _sc_gather_fn(B, S, H):
    mesh = plsc.VectorSubcoreMesh(
        core_axis_name="c", subcore_axis_name="s", num_cores=NC, num_subcores=NS)
    ex_per_w = B // NW
    nch = S // CH

    @functools.partial(
        pl.kernel,
        out_type=(
            jax.ShapeDtypeStruct((B * S, H), jnp.float32),
            jax.ShapeDtypeStruct((B, H), jnp.float32),
            jax.ShapeDtypeStruct((B, H), jnp.float32),
        ),
        mesh=mesh,
        scratch_types=[
            pltpu.VMEM((S,), jnp.int32),
            pltpu.VMEM((NBUF, CH, H), jnp.float32),
            pltpu.VMEM((1, H), jnp.float32),
            pltpu.VMEM((16,), jnp.int32),
            pltpu.VMEM((16, H), jnp.float32),
            pltpu.SemaphoreType.DMA,
            pltpu.SemaphoreType.DMA,
            pltpu.SemaphoreType.DMA,
            pltpu.SemaphoreType.DMA,
            pltpu.SemaphoreType.DMA,
            pltpu.SemaphoreType.DMA,
            pltpu.SemaphoreType.DMA,
            pltpu.SemaphoreType.DMA,
        ],
    )
    def k(ids_hbm, antidx_hbm, annidx_hbm, wtab, anttab, anntab,
          g_out, antrows, annsums,
          idx_ex, rows_v, acc_v, idx16_v, rows16_v,
          gsem0, gsem1, gsem2, gsem3, wsem0, wsem1, wsem2, wsem3):
        wid = lax.axis_index("s") * NC + lax.axis_index("c")
        nj = H // 16
        gsem = (gsem0, gsem1, gsem2, gsem3)
        wsem = (wsem0, wsem1, wsem2, wsem3)
        wbytes = CH * H * 4

        for e in range(ex_per_w):
            b = wid * ex_per_w + e
            pltpu.sync_copy(ids_hbm.at[pl.ds(b * S, S)], idx_ex)

            def issue_gather(c, k_):
                pltpu.async_copy(
                    wtab.at[idx_ex.at[pl.ds(c * CH, CH)]],
                    rows_v.at[k_], gsem[k_])

            def drain_and_write(c, k_, b=b):
                pltpu.make_async_copy(
                    wtab.at[idx_ex.at[pl.ds(0, CH)]], rows_v.at[k_],
                    gsem[k_]).wait()
                pltpu.async_copy(
                    rows_v.at[k_], g_out.at[pl.ds(b * S + c * CH, CH)],
                    wsem[k_])

            for k_ in range(NBUF):
                issue_gather(k_, k_)
            for k_ in range(NBUF):
                drain_and_write(k_, k_)

            def group_body(i, carry, b=b):
                for k_ in range(NBUF):
                    c = i * NBUF + k_
                    pltpu.make_async_copy(
                        rows_v.at[k_], g_out.at[pl.ds(b * S, CH)],
                        wsem[k_]).wait()
                    issue_gather(c, k_)
                for k_ in range(NBUF):
                    drain_and_write(i * NBUF + k_, k_)
                return carry

            lax.fori_loop(1, nch // NBUF, group_body, 0)
            for k_ in range(NBUF):
                pltpu.make_async_copy(
                    rows_v.at[k_], g_out.at[pl.ds(b * S, CH)], wsem[k_]).wait()

            pltpu.sync_copy(antidx_hbm.at[pl.ds(b * 16, 16)], idx16_v)
            pltpu.async_copy(anttab.at[idx16_v], rows16_v, gsem0).wait()
            pltpu.sync_copy(rows16_v.at[pl.ds(0, 1)], antrows.at[pl.ds(b, 1)])

            zf32 = jnp.zeros((16,), jnp.float32)
            for j in range(nj):
                acc_v[0, pl.ds(j * 16, 16)] = zf32
            pltpu.sync_copy(annidx_hbm.at[pl.ds(b * 16, 16)], idx16_v)
            pltpu.async_copy(anntab.at[idx16_v], rows16_v, gsem0).wait()

            def row_body(r, carry):
                for j in range(nj):
                    plsc.addupdate(acc_v.at[0, pl.ds(j * 16, 16)],
                                   rows16_v[r, pl.ds(j * 16, 16)])
                return carry

            lax.fori_loop(0, 16, row_body, 0)
            pltpu.sync_copy(acc_v, annsums.at[pl.ds(b, 1)])

    return k


def _fused_body(g_ref, pos_ref, antrows_ref, annsums_ref,
                sentw_ref, antw_ref, annw_ref, gamma_ref, lnbeta_ref,
                out_ref, u_ref, v_ref, pm_ref, am_ref, *, S, L):
    b = pl.program_id(0)

    @pl.when(b == 0)
    def _precompute():
        pm_ref[...] = jnp.mean(pos_ref[...], axis=0, keepdims=True)
        am_ref[...] = annsums_ref[...] * (1.0 / L)
        dt = (((1,), (1,)), ((), ()))
        dp = (((1,), (0,)), ((), ()))
        u_ref[...] = lax.dot_general(
            lax.dot_general(antrows_ref[...], antw_ref[...], dt,
                            preferred_element_type=jnp.float32),
            sentw_ref[...], dp, preferred_element_type=jnp.float32)
        v_ref[...] = lax.dot_general(
            lax.dot_general(am_ref[...], annw_ref[...], dt,
                            preferred_element_type=jnp.float32),
            sentw_ref[...], dp, preferred_element_type=jnp.float32)

    g = g_ref[...]
    m = jnp.sum(g, axis=0, keepdims=True) * (1.0 / S) + pm_ref[...]
    alpha = jnp.sum(m * u_ref[pl.ds(b, 1)], axis=1, keepdims=True)
    beta = jnp.sum(m * v_ref[pl.ds(b, 1)], axis=1, keepdims=True)
    addvec = alpha * antrows_ref[pl.ds(b, 1)] + beta * am_ref[pl.ds(b, 1)]
    x = g + pos_ref[...] + addvec
    mu = jnp.mean(x, axis=1, keepdims=True)
    xc = x - mu
    var = jnp.mean(xc * xc, axis=1, keepdims=True)
    out_ref[...] = xc * lax.rsqrt(var + LN_EPS) * gamma_ref[...] + lnbeta_ref[...]


def kernel(input_ids, annotator_ids, annotations, word_emb, pos_emb, sent_W,
           annotator_W, annotation_W, annotator_table, annotation_table,
           ln_gamma, ln_beta):
    B, S = input_ids.shape
    V, H = word_emb.shape
    L = annotations.shape[1]

    ids_flat = input_ids.reshape(-1).astype(jnp.int32)
    ant_idx16 = jnp.concatenate(
        [annotator_ids[:, None].astype(jnp.int32),
         jnp.zeros((B, 15), jnp.int32)], axis=1).reshape(-1)
    ann_idx16 = jnp.concatenate(
        [annotations.astype(jnp.int32),
         jnp.zeros((B, 16 - L), jnp.int32)], axis=1).reshape(-1)

    sc = _sc_gather_fn(B, S, H)
    g, antrows, annsums = sc(
        ids_flat, ant_idx16, ann_idx16, word_emb, annotator_table,
        annotation_table)

    c0 = lambda b: (0, 0)
    out = pl.pallas_call(
        functools.partial(_fused_body, S=S, L=L),
        grid=(B,),
        in_specs=[
            pl.BlockSpec((S, H), lambda b: (b, 0)),
            pl.BlockSpec((S, H), c0),
            pl.BlockSpec((B, H), c0),
            pl.BlockSpec((B, H), c0),
            pl.BlockSpec((H, H), c0),
            pl.BlockSpec((H, H), c0),
            pl.BlockSpec((H, H), c0),
            pl.BlockSpec((1, H), c0),
            pl.BlockSpec((1, H), c0),
        ],
        out_specs=pl.BlockSpec((S, H), lambda b: (b, 0)),
        out_shape=jax.ShapeDtypeStruct((B * S, H), jnp.float32),
        scratch_shapes=[
            pltpu.VMEM((B, H), jnp.float32),
            pltpu.VMEM((B, H), jnp.float32),
            pltpu.VMEM((1, H), jnp.float32),
            pltpu.VMEM((B, H), jnp.float32),
        ],
        compiler_params=pltpu.CompilerParams(
            dimension_semantics=("arbitrary",)),
    )(g, pos_emb, antrows, annsums, sent_W, annotator_W, annotation_W,
      ln_gamma.reshape(1, H), ln_beta.reshape(1, H))

    return out.reshape(B, S, H)


# speedup vs baseline: 1.0133x; 1.0133x over previous
import functools

import jax
import jax.numpy as jnp
from jax import lax
from jax.experimental import pallas as pl
from jax.experimental.pallas import tpu as pltpu
from jax.experimental.pallas import tpu_sc as plsc

LN_EPS = 1e-7
NC = 2
NS = 16
NW = NC * NS
CH = 32
NBUF = 4


def _sc_gather_fn(B, S, H):
    mesh = plsc.VectorSubcoreMesh(
        core_axis_name="c", subcore_axis_name="s", num_cores=NC, num_subcores=NS)
    ex_per_w = B // NW
    nch = S // CH

    @functools.partial(
        pl.kernel,
        out_type=(
            jax.ShapeDtypeStruct((B * S, H), jnp.float32),
            jax.ShapeDtypeStruct((B, H), jnp.float32),
            jax.ShapeDtypeStruct((B, H), jnp.float32),
        ),
        mesh=mesh,
        scratch_types=[
            pltpu.VMEM((S,), jnp.int32),
            pltpu.VMEM((NBUF, CH, H), jnp.float32),
            pltpu.VMEM((1, H), jnp.float32),
            pltpu.VMEM((16,), jnp.int32),
            pltpu.VMEM((16, H), jnp.float32),
            pltpu.SemaphoreType.DMA,
            pltpu.SemaphoreType.DMA,
            pltpu.SemaphoreType.DMA,
            pltpu.SemaphoreType.DMA,
            pltpu.SemaphoreType.DMA,
            pltpu.SemaphoreType.DMA,
            pltpu.SemaphoreType.DMA,
            pltpu.SemaphoreType.DMA,
        ],
    )
    def k(ids_hbm, antidx_hbm, annidx_hbm, wtab, anttab, anntab,
          g_out, antrows, annsums,
          idx_ex, rows_v, acc_v, idx16_v, rows16_v,
          gsem0, gsem1, gsem2, gsem3, wsem0, wsem1, wsem2, wsem3):
        wid = lax.axis_index("s") * NC + lax.axis_index("c")
        nj = H // 16
        gsem = (gsem0, gsem1, gsem2, gsem3)
        wsem = (wsem0, wsem1, wsem2, wsem3)
        wbytes = CH * H * 4

        for e in range(ex_per_w):
            b = wid * ex_per_w + e
            pltpu.sync_copy(ids_hbm.at[pl.ds(b * S, S)], idx_ex)

            def issue_gather(c, k_):
                pltpu.async_copy(
                    wtab.at[idx_ex.at[pl.ds(c * CH, CH)]],
                    rows_v.at[k_], gsem[k_])

            def gwait(k_):
                pltpu.make_async_copy(
                    wtab.at[idx_ex.at[pl.ds(0, CH)]], rows_v.at[k_],
                    gsem[k_]).wait()

            def issue_write(c, k_, b=b):
                pltpu.async_copy(
                    rows_v.at[k_], g_out.at[pl.ds(b * S + c * CH, CH)],
                    wsem[k_])

            def wwait(k_, b=b):
                pltpu.make_async_copy(
                    rows_v.at[k_], g_out.at[pl.ds(b * S, CH)], wsem[k_]).wait()

            issue_gather(0, 0)
            for k_ in range(1, NBUF):
                issue_gather(k_, k_)
                gwait(k_ - 1)
                issue_write(k_ - 1, k_ - 1)

            def steady_body(i, carry):
                for k_ in range(NBUF):
                    c = i * NBUF + k_
                    wwait(k_)
                    issue_gather(c, k_)
                    gwait((k_ + NBUF - 1) % NBUF)
                    issue_write(c - 1, (k_ + NBUF - 1) % NBUF)
                return carry

            lax.fori_loop(1, nch // NBUF, steady_body, 0)
            gwait(NBUF - 1)
            issue_write(nch - 1, NBUF - 1)
            for k_ in range(NBUF):
                wwait(k_)

            pltpu.sync_copy(antidx_hbm.at[pl.ds(b * 16, 16)], idx16_v)
            pltpu.async_copy(anttab.at[idx16_v], rows16_v, gsem0).wait()
            pltpu.sync_copy(rows16_v.at[pl.ds(0, 1)], antrows.at[pl.ds(b, 1)])

            zf32 = jnp.zeros((16,), jnp.float32)
            for j in range(nj):
                acc_v[0, pl.ds(j * 16, 16)] = zf32
            pltpu.sync_copy(annidx_hbm.at[pl.ds(b * 16, 16)], idx16_v)
            pltpu.async_copy(anntab.at[idx16_v], rows16_v, gsem0).wait()

            def row_body(r, carry):
                for j in range(nj):
                    plsc.addupdate(acc_v.at[0, pl.ds(j * 16, 16)],
                                   rows16_v[r, pl.ds(j * 16, 16)])
                return carry

            lax.fori_loop(0, 16, row_body, 0)
            pltpu.sync_copy(acc_v, annsums.at[pl.ds(b, 1)])

    return k


def _fused_body(g_ref, pos_ref, antrows_ref, annsums_ref,
                sentw_ref, antw_ref, annw_ref, gamma_ref, lnbeta_ref,
                out_ref, u_ref, v_ref, pm_ref, am_ref, *, S, L):
    b = pl.program_id(0)

    @pl.when(b == 0)
    def _precompute():
        pm_ref[...] = jnp.mean(pos_ref[...], axis=0, keepdims=True)
        am_ref[...] = annsums_ref[...] * (1.0 / L)
        dt = (((1,), (1,)), ((), ()))
        dp = (((1,), (0,)), ((), ()))
        u_ref[...] = lax.dot_general(
            lax.dot_general(antrows_ref[...], antw_ref[...], dt,
                            preferred_element_type=jnp.float32),
            sentw_ref[...], dp, preferred_element_type=jnp.float32)
        v_ref[...] = lax.dot_general(
            lax.dot_general(am_ref[...], annw_ref[...], dt,
                            preferred_element_type=jnp.float32),
            sentw_ref[...], dp, preferred_element_type=jnp.float32)

    g = g_ref[...]
    m = jnp.sum(g, axis=0, keepdims=True) * (1.0 / S) + pm_ref[...]
    alpha = jnp.sum(m * u_ref[pl.ds(b, 1)], axis=1, keepdims=True)
    beta = jnp.sum(m * v_ref[pl.ds(b, 1)], axis=1, keepdims=True)
    addvec = alpha * antrows_ref[pl.ds(b, 1)] + beta * am_ref[pl.ds(b, 1)]
    x = g + pos_ref[...] + addvec
    mu = jnp.mean(x, axis=1, keepdims=True)
    xc = x - mu
    var = jnp.mean(xc * xc, axis=1, keepdims=True)
    out_ref[...] = xc * lax.rsqrt(var + LN_EPS) * gamma_ref[...] + lnbeta_ref[...]


def kernel(input_ids, annotator_ids, annotations, word_emb, pos_emb, sent_W,
           annotator_W, annotation_W, annotator_table, annotation_table,
           ln_gamma, ln_beta):
    B, S = input_ids.shape
    V, H = word_emb.shape
    L = annotations.shape[1]

    ids_flat = input_ids.reshape(-1).astype(jnp.int32)
    ant_idx16 = jnp.concatenate(
        [annotator_ids[:, None].astype(jnp.int32),
         jnp.zeros((B, 15), jnp.int32)], axis=1).reshape(-1)
    ann_idx16 = jnp.concatenate(
        [annotations.astype(jnp.int32),
         jnp.zeros((B, 16 - L), jnp.int32)], axis=1).reshape(-1)

    sc = _sc_gather_fn(B, S, H)
    g, antrows, annsums = sc(
        ids_flat, ant_idx16, ann_idx16, word_emb, annotator_table,
        annotation_table)

    c0 = lambda b: (0, 0)
    out = pl.pallas_call(
        functools.partial(_fused_body, S=S, L=L),
        grid=(B,),
        in_specs=[
            pl.BlockSpec((S, H), lambda b: (b, 0)),
            pl.BlockSpec((S, H), c0),
            pl.BlockSpec((B, H), c0),
            pl.BlockSpec((B, H), c0),
            pl.BlockSpec((H, H), c0),
            pl.BlockSpec((H, H), c0),
            pl.BlockSpec((H, H), c0),
            pl.BlockSpec((1, H), c0),
            pl.BlockSpec((1, H), c0),
        ],
        out_specs=pl.BlockSpec((S, H), lambda b: (b, 0)),
        out_shape=jax.ShapeDtypeStruct((B * S, H), jnp.float32),
        scratch_shapes=[
            pltpu.VMEM((B, H), jnp.float32),
            pltpu.VMEM((B, H), jnp.float32),
            pltpu.VMEM((1, H), jnp.float32),
            pltpu.VMEM((B, H), jnp.float32),
        ],
        compiler_params=pltpu.CompilerParams(
            dimension_semantics=("arbitrary",)),
    )(g, pos_emb, antrows, annsums, sent_W, annotator_W, annotation_W,
      ln_gamma.reshape(1, H), ln_beta.reshape(1, H))

    return out.reshape(B, S, H)


# speedup vs baseline: 1.0665x; 1.0525x over previous
import functools

import jax
import jax.numpy as jnp
from jax import lax
from jax.experimental import pallas as pl
from jax.experimental.pallas import tpu as pltpu
from jax.experimental.pallas import tpu_sc as plsc

LN_EPS = 1e-7
NC = 2
NS = 16
NW = NC * NS
CH = 32
NBUF = 4


def _sc_gather_fn(B, S, H):
    mesh = plsc.VectorSubcoreMesh(
        core_axis_name="c", subcore_axis_name="s", num_cores=NC, num_subcores=NS)
    ex_per_w = B // NW
    nch = S // CH

    @functools.partial(
        pl.kernel,
        out_type=(
            jax.ShapeDtypeStruct((B * S, H), jnp.float32),
            jax.ShapeDtypeStruct((B, H), jnp.float32),
            jax.ShapeDtypeStruct((B, H), jnp.float32),
        ),
        mesh=mesh,
        scratch_types=[
            pltpu.VMEM((S,), jnp.int32),
            pltpu.VMEM((NBUF, CH, H), jnp.float32),
            pltpu.VMEM((1, H), jnp.float32),
            pltpu.VMEM((16,), jnp.int32),
            pltpu.VMEM((16, H), jnp.float32),
            pltpu.SemaphoreType.DMA,
            pltpu.SemaphoreType.DMA,
            pltpu.SemaphoreType.DMA,
            pltpu.SemaphoreType.DMA,
            pltpu.SemaphoreType.DMA,
            pltpu.SemaphoreType.DMA,
            pltpu.SemaphoreType.DMA,
            pltpu.SemaphoreType.DMA,
        ],
    )
    def k(ids_hbm, antidx_hbm, annidx_hbm, wtab, anttab, anntab,
          g_out, antrows, annsums,
          idx_ex, rows_v, acc_v, idx16_v, rows16_v,
          gsem0, gsem1, gsem2, gsem3, wsem0, wsem1, wsem2, wsem3):
        wid = lax.axis_index("s") * NC + lax.axis_index("c")
        nj = H // 16
        gsem = (gsem0, gsem1, gsem2, gsem3)
        wsem = (wsem0, wsem1, wsem2, wsem3)
        wbytes = CH * H * 4

        for e in range(ex_per_w):
            b = wid * ex_per_w + e
            pltpu.sync_copy(ids_hbm.at[pl.ds(b * S, S)], idx_ex)

            def issue_gather(c, k_):
                pltpu.async_copy(
                    wtab.at[idx_ex.at[pl.ds(c * CH, CH)]],
                    rows_v.at[k_], gsem[k_])

            def gwait(k_):
                pltpu.make_async_copy(
                    wtab.at[idx_ex.at[pl.ds(0, CH)]], rows_v.at[k_],
                    gsem[k_]).wait()

            def issue_write(c, k_, b=b):
                pltpu.async_copy(
                    rows_v.at[k_], g_out.at[pl.ds(b * S + c * CH, CH)],
                    wsem[k_])

            def wwait(k_, b=b):
                pltpu.make_async_copy(
                    rows_v.at[k_], g_out.at[pl.ds(b * S, CH)], wsem[k_]).wait()

            issue_gather(0, 0)
            for k_ in range(1, NBUF):
                issue_gather(k_, k_)
                gwait(k_ - 1)
                issue_write(k_ - 1, k_ - 1)

            def steady_body(i, carry):
                for k_ in range(NBUF):
                    c = i * NBUF + k_
                    wwait(k_)
                    issue_gather(c, k_)
                    gwait((k_ + NBUF - 1) % NBUF)
                    issue_write(c - 1, (k_ + NBUF - 1) % NBUF)
                return carry

            lax.fori_loop(1, nch // NBUF, steady_body, 0)
            gwait(NBUF - 1)
            issue_write(nch - 1, NBUF - 1)
            for k_ in range(NBUF):
                wwait(k_)

            pltpu.sync_copy(antidx_hbm.at[pl.ds(b * 16, 16)], idx16_v)
            pltpu.async_copy(anttab.at[idx16_v], rows16_v, gsem0).wait()
            pltpu.sync_copy(rows16_v.at[pl.ds(0, 1)], antrows.at[pl.ds(b, 1)])

            zf32 = jnp.zeros((16,), jnp.float32)
            for j in range(nj):
                acc_v[0, pl.ds(j * 16, 16)] = zf32
            pltpu.sync_copy(annidx_hbm.at[pl.ds(b * 16, 16)], idx16_v)
            pltpu.async_copy(anntab.at[idx16_v], rows16_v, gsem0).wait()

            def row_body(r, carry):
                for j in range(nj):
                    plsc.addupdate(acc_v.at[0, pl.ds(j * 16, 16)],
                                   rows16_v[r, pl.ds(j * 16, 16)])
                return carry

            lax.fori_loop(0, 16, row_body, 0)
            pltpu.sync_copy(acc_v, annsums.at[pl.ds(b, 1)])

    return k


def _fused_body(g_ref, pos_ref, antrows_ref, annsums_ref,
                sentw_ref, antw_ref, annw_ref, gamma_ref, lnbeta_ref,
                prev_ref, out_ref, u_ref, v_ref, pm_ref, am_ref, *, S, L):
    del prev_ref
    b = pl.program_id(0)

    @pl.when(b == 0)
    def _precompute():
        pm_ref[...] = jnp.mean(pos_ref[...], axis=0, keepdims=True)
        am_ref[...] = annsums_ref[...] * (1.0 / L)
        dt = (((1,), (1,)), ((), ()))
        dp = (((1,), (0,)), ((), ()))
        u_ref[...] = lax.dot_general(
            lax.dot_general(antrows_ref[...], antw_ref[...], dt,
                            preferred_element_type=jnp.float32),
            sentw_ref[...], dp, preferred_element_type=jnp.float32)
        v_ref[...] = lax.dot_general(
            lax.dot_general(am_ref[...], annw_ref[...], dt,
                            preferred_element_type=jnp.float32),
            sentw_ref[...], dp, preferred_element_type=jnp.float32)

    g = g_ref[...]
    m = jnp.sum(g, axis=0, keepdims=True) * (1.0 / S) + pm_ref[...]
    alpha = jnp.sum(m * u_ref[pl.ds(b, 1)], axis=1, keepdims=True)
    beta = jnp.sum(m * v_ref[pl.ds(b, 1)], axis=1, keepdims=True)
    addvec = alpha * antrows_ref[pl.ds(b, 1)] + beta * am_ref[pl.ds(b, 1)]
    x = g + pos_ref[...] + addvec
    mu = jnp.mean(x, axis=1, keepdims=True)
    xc = x - mu
    var = jnp.mean(xc * xc, axis=1, keepdims=True)
    out_ref[...] = xc * lax.rsqrt(var + LN_EPS) * gamma_ref[...] + lnbeta_ref[...]


def kernel(input_ids, annotator_ids, annotations, word_emb, pos_emb, sent_W,
           annotator_W, annotation_W, annotator_table, annotation_table,
           ln_gamma, ln_beta):
    B, S = input_ids.shape
    V, H = word_emb.shape
    L = annotations.shape[1]

    ids_flat = input_ids.reshape(-1).astype(jnp.int32)
    ant_idx16 = jnp.concatenate(
        [annotator_ids[:, None].astype(jnp.int32),
         jnp.zeros((B, 15), jnp.int32)], axis=1).reshape(-1)
    ann_idx16 = jnp.concatenate(
        [annotations.astype(jnp.int32),
         jnp.zeros((B, 16 - L), jnp.int32)], axis=1).reshape(-1)

    Bh = B // 2
    sc = _sc_gather_fn(Bh, S, H)
    halves = []
    for h in range(2):
        halves.append(sc(
            ids_flat[h * Bh * S:(h + 1) * Bh * S],
            ant_idx16[h * Bh * 16:(h + 1) * Bh * 16],
            ann_idx16[h * Bh * 16:(h + 1) * Bh * 16],
            word_emb, annotator_table, annotation_table))

    gamma2 = ln_gamma.reshape(1, H)
    beta2 = ln_beta.reshape(1, H)
    c0 = lambda b: (0, 0)
    prev = None
    for h in range(2):
        g, antrows, annsums = halves[h]
        if prev is None:
            prev = jnp.zeros((8, 128), jnp.float32)
            alias = {}
            prev_spec = pl.BlockSpec(memory_space=pl.ANY)
        else:
            alias = {9: 0}
            prev_spec = pl.BlockSpec(memory_space=pl.ANY)
        prev = pl.pallas_call(
            functools.partial(_fused_body, S=S, L=L),
            grid=(Bh,),
            in_specs=[
                pl.BlockSpec((S, H), lambda b: (b, 0)),
                pl.BlockSpec((S, H), c0),
                pl.BlockSpec((Bh, H), c0),
                pl.BlockSpec((Bh, H), c0),
                pl.BlockSpec((H, H), c0),
                pl.BlockSpec((H, H), c0),
                pl.BlockSpec((H, H), c0),
                pl.BlockSpec((1, H), c0),
                pl.BlockSpec((1, H), c0),
                prev_spec,
            ],
            out_specs=pl.BlockSpec(
                (S, H), lambda b, h=h: (b + h * Bh, 0)),
            out_shape=jax.ShapeDtypeStruct((B * S, H), jnp.float32),
            scratch_shapes=[
                pltpu.VMEM((Bh, H), jnp.float32),
                pltpu.VMEM((Bh, H), jnp.float32),
                pltpu.VMEM((1, H), jnp.float32),
                pltpu.VMEM((Bh, H), jnp.float32),
            ],
            input_output_aliases=alias,
            compiler_params=pltpu.CompilerParams(
                dimension_semantics=("arbitrary",)),
        )(g, pos_emb, antrows, annsums, sent_W, annotator_W, annotation_W,
          gamma2, beta2, prev)

    return prev.reshape(B, S, H)


# speedup vs baseline: 1.1702x; 1.0972x over previous
import functools

import jax
import jax.numpy as jnp
from jax import lax
from jax.experimental import pallas as pl
from jax.experimental.pallas import tpu as pltpu
from jax.experimental.pallas import tpu_sc as plsc

LN_EPS = 1e-7
NC = 2
NS = 16
NW = NC * NS
CH = 16
NBUF = 4


def _sc_gather_fn(B, S, H):
    mesh = plsc.VectorSubcoreMesh(
        core_axis_name="c", subcore_axis_name="s", num_cores=NC, num_subcores=NS)
    ex_per_w = B // NW
    nch = S // CH

    S2 = S // 2
    npair = S2 // CH

    @functools.partial(
        pl.kernel,
        out_type=(
            jax.ShapeDtypeStruct((B * S2, H), jnp.uint32),
            jax.ShapeDtypeStruct((B, H), jnp.float32),
            jax.ShapeDtypeStruct((B, H), jnp.float32),
        ),
        mesh=mesh,
        compiler_params=pltpu.CompilerParams(needs_layout_passes=False),
        scratch_types=[
            pltpu.VMEM((S,), jnp.int32),
            pltpu.VMEM((4, CH, H), jnp.float32),
            pltpu.VMEM((2, CH, H), jnp.uint32),
            pltpu.VMEM((1, H), jnp.float32),
            pltpu.VMEM((16,), jnp.int32),
            pltpu.SemaphoreType.DMA,
            pltpu.SemaphoreType.DMA,
            pltpu.SemaphoreType.DMA,
            pltpu.SemaphoreType.DMA,
            pltpu.SemaphoreType.DMA,
            pltpu.SemaphoreType.DMA,
        ],
    )
    def k(ids_hbm, antidx_hbm, annidx_hbm, wtab, anttab, anntab,
          g_out, antrows, annsums,
          idx_ex, rows_v, brows_v, acc_v, idx16_v,
          gsem0, gsem1, gsem2, gsem3, wsem0, wsem1):
        wid = lax.axis_index("s") * NC + lax.axis_index("c")
        nj = H // 16
        gsem = (gsem0, gsem1, gsem2, gsem3)
        wsem = (wsem0, wsem1)

        for e in range(ex_per_w):
            b = wid * ex_per_w + e
            pltpu.sync_copy(ids_hbm.at[pl.ds(b * S, S)], idx_ex)

            def issue_pair(p, k2, b=b):
                pltpu.async_copy(
                    wtab.at[idx_ex.at[pl.ds(p * CH, CH)]],
                    rows_v.at[2 * k2], gsem[2 * k2])
                pltpu.async_copy(
                    wtab.at[idx_ex.at[pl.ds(S2 + p * CH, CH)]],
                    rows_v.at[2 * k2 + 1], gsem[2 * k2 + 1])

            def gwait_pair(k2):
                for d in range(2):
                    pltpu.make_async_copy(
                        wtab.at[idx_ex.at[pl.ds(0, CH)]],
                        rows_v.at[2 * k2 + d], gsem[2 * k2 + d]).wait()

            def wwait(k2, b=b):
                pltpu.make_async_copy(
                    brows_v.at[k2], g_out.at[pl.ds(b * S2, CH)],
                    wsem[k2]).wait()

            def pack_pair(k2):
                lo = 2 * k2
                hi = 2 * k2 + 1

                def prow(r, carry):
                    for j in range(nj):
                        a = rows_v[lo, r, pl.ds(j * 16, 16)]
                        bb = rows_v[hi, r, pl.ds(j * 16, 16)]
                        brows_v[k2, r, pl.ds(j * 16, 16)] = plsc.bitcast(
                            plsc.pack(a, bb,
                                      format=plsc.PackFormat.INTERLEAVED),
                            jnp.uint32)
                    return carry
                lax.fori_loop(0, CH, prow, 0)

            def process(p, k2, refill, do_wwait, b=b):
                gwait_pair(k2)
                if do_wwait:
                    wwait(k2)
                pack_pair(k2)
                pltpu.async_copy(
                    brows_v.at[k2], g_out.at[pl.ds(b * S2 + p * CH, CH)],
                    wsem[k2])
                if refill:
                    issue_pair(p + 2, k2)

            issue_pair(0, 0)
            issue_pair(1, 1)
            process(0, 0, True, False)
            process(1, 1, True, False)

            def steady_body(i, carry):
                process(2 * i, 0, True, True)
                process(2 * i + 1, 1, True, True)
                return carry

            lax.fori_loop(1, npair // 2 - 1, steady_body, 0)
            process(npair - 2, 0, False, True)
            process(npair - 1, 1, False, True)
            wwait(0)
            wwait(1)

            pltpu.sync_copy(antidx_hbm.at[pl.ds(b * 16, 16)], idx16_v)
            pltpu.async_copy(
                anttab.at[idx16_v], rows_v.at[0], gsem0).wait()
            pltpu.sync_copy(rows_v.at[0].at[pl.ds(0, 1)],
                            antrows.at[pl.ds(b, 1)])

            zf32 = jnp.zeros((16,), jnp.float32)
            for j in range(nj):
                acc_v[0, pl.ds(j * 16, 16)] = zf32
            pltpu.sync_copy(annidx_hbm.at[pl.ds(b * 16, 16)], idx16_v)
            pltpu.async_copy(
                anntab.at[idx16_v], rows_v.at[0], gsem0).wait()

            def row_body(r, carry):
                for j in range(nj):
                    plsc.addupdate(acc_v.at[0, pl.ds(j * 16, 16)],
                                   rows_v[0, r, pl.ds(j * 16, 16)])
                return carry

            lax.fori_loop(0, 16, row_body, 0)
            pltpu.sync_copy(acc_v, annsums.at[pl.ds(b, 1)])

    return k


def _fused_body(g_ref, pos_ref, antrows_ref, annsums_ref,
                sentw_ref, antw_ref, annw_ref, gamma_ref, lnbeta_ref,
                prev_ref, out_ref, u_ref, v_ref, pm_ref, am_ref, *, S, L):
    del prev_ref
    b = pl.program_id(0)

    @pl.when(b == 0)
    def _precompute():
        pm_ref[...] = jnp.mean(pos_ref[...], axis=0, keepdims=True)
        am_ref[...] = annsums_ref[...] * (1.0 / L)
        dt = (((1,), (1,)), ((), ()))
        dp = (((1,), (0,)), ((), ()))
        u_ref[...] = lax.dot_general(
            lax.dot_general(antrows_ref[...], antw_ref[...], dt,
                            preferred_element_type=jnp.float32),
            sentw_ref[...], dp, preferred_element_type=jnp.float32)
        v_ref[...] = lax.dot_general(
            lax.dot_general(am_ref[...], annw_ref[...], dt,
                            preferred_element_type=jnp.float32),
            sentw_ref[...], dp, preferred_element_type=jnp.float32)

    w = g_ref[...]
    lo = lax.bitcast_convert_type(w << jnp.uint32(16), jnp.float32)
    hi = lax.bitcast_convert_type(w & jnp.uint32(0xFFFF0000), jnp.float32)
    g = jnp.concatenate([lo, hi], axis=0)
    m = jnp.sum(g, axis=0, keepdims=True) * (1.0 / S) + pm_ref[...]
    alpha = jnp.sum(m * u_ref[pl.ds(b, 1)], axis=1, keepdims=True)
    beta = jnp.sum(m * v_ref[pl.ds(b, 1)], axis=1, keepdims=True)
    addvec = alpha * antrows_ref[pl.ds(b, 1)] + beta * am_ref[pl.ds(b, 1)]
    x = g + pos_ref[...] + addvec
    mu = jnp.mean(x, axis=1, keepdims=True)
    xc = x - mu
    var = jnp.mean(xc * xc, axis=1, keepdims=True)
    out_ref[...] = xc * lax.rsqrt(var + LN_EPS) * gamma_ref[...] + lnbeta_ref[...]


def kernel(input_ids, annotator_ids, annotations, word_emb, pos_emb, sent_W,
           annotator_W, annotation_W, annotator_table, annotation_table,
           ln_gamma, ln_beta):
    B, S = input_ids.shape
    V, H = word_emb.shape
    L = annotations.shape[1]

    ids_flat = input_ids.reshape(-1).astype(jnp.int32)
    ant_idx16 = jnp.concatenate(
        [annotator_ids[:, None].astype(jnp.int32),
         jnp.zeros((B, 15), jnp.int32)], axis=1).reshape(-1)
    ann_idx16 = jnp.concatenate(
        [annotations.astype(jnp.int32),
         jnp.zeros((B, 16 - L), jnp.int32)], axis=1).reshape(-1)

    Bh = B // 2
    sc = _sc_gather_fn(Bh, S, H)
    halves = []
    for h in range(2):
        halves.append(sc(
            ids_flat[h * Bh * S:(h + 1) * Bh * S],
            ant_idx16[h * Bh * 16:(h + 1) * Bh * 16],
            ann_idx16[h * Bh * 16:(h + 1) * Bh * 16],
            word_emb, annotator_table, annotation_table))

    gamma2 = ln_gamma.reshape(1, H)
    beta2 = ln_beta.reshape(1, H)
    c0 = lambda b: (0, 0)
    prev = None
    for h in range(2):
        g, antrows, annsums = halves[h]
        if prev is None:
            prev = jnp.zeros((8, 128), jnp.float32)
            alias = {}
            prev_spec = pl.BlockSpec(memory_space=pl.ANY)
        else:
            alias = {9: 0}
            prev_spec = pl.BlockSpec(memory_space=pl.ANY)
        prev = pl.pallas_call(
            functools.partial(_fused_body, S=S, L=L),
            grid=(Bh,),
            in_specs=[
                pl.BlockSpec((S // 2, H), lambda b: (b, 0)),
                pl.BlockSpec((S, H), c0),
                pl.BlockSpec((Bh, H), c0),
                pl.BlockSpec((Bh, H), c0),
                pl.BlockSpec((H, H), c0),
                pl.BlockSpec((H, H), c0),
                pl.BlockSpec((H, H), c0),
                pl.BlockSpec((1, H), c0),
                pl.BlockSpec((1, H), c0),
                prev_spec,
            ],
            out_specs=pl.BlockSpec(
                (S, H), lambda b, h=h: (b + h * Bh, 0)),
            out_shape=jax.ShapeDtypeStruct((B * S, H), jnp.float32),
            scratch_shapes=[
                pltpu.VMEM((Bh, H), jnp.float32),
                pltpu.VMEM((Bh, H), jnp.float32),
                pltpu.VMEM((1, H), jnp.float32),
                pltpu.VMEM((Bh, H), jnp.float32),
            ],
            input_output_aliases=alias,
            compiler_params=pltpu.CompilerParams(
                dimension_semantics=("arbitrary",)),
        )(g, pos_emb, antrows, annsums, sent_W, annotator_W, annotation_W,
          gamma2, beta2, prev)

    return prev.reshape(B, S, H)
